# baseline (speedup 1.0000x reference)
import functools

import jax
import jax.numpy as jnp
from jax import lax
from jax.experimental import pallas as pl
from jax.experimental.pallas import tpu as pltpu
from jax.experimental.pallas import tpu_sc as plsc

B = 16384
V = 8
L = 16
NC, NS = 2, 16
NW = NC * NS
BPW = B // NW
CHUNK = 128
NCH = BPW // CHUNK
GRP = BPW // L


@functools.lru_cache(maxsize=1)
def _build():
    mesh = plsc.VectorSubcoreMesh(
        core_axis_name="c", subcore_axis_name="s", num_cores=NC, num_subcores=NS
    )

    @functools.partial(
        pl.kernel,
        out_type=jax.ShapeDtypeStruct((NW, NCH, CHUNK), jnp.int32),
        mesh=mesh,
        scratch_types=[
            pltpu.VMEM((NCH, CHUNK), jnp.int32),
            pltpu.VMEM((NCH, CHUNK), jnp.int32),
            pltpu.VMEM((NCH, CHUNK), jnp.int32),
            pltpu.VMEM((V, NCH, CHUNK), jnp.int32),
            pltpu.VMEM((NCH, CHUNK), jnp.int32),
            pltpu.SemaphoreType.DMA,
        ],
        compiler_params=pltpu.CompilerParams(
            needs_layout_passes=False, use_tc_tiling_on_sc=False
        ),
    )
    def sc_kernel(tok_hbm, tti_hbm, wclt_hbm, out_hbm,
                  tok_v, ti_v, sidx_v, cols_v, out_v, sem):
        wid = lax.axis_index("s") * NC + lax.axis_index("c")
        pltpu.sync_copy(tok_hbm.at[wid], tok_v)

        cps = [pltpu.async_copy(tti_hbm.at[tok_v.at[jnp.int32(j)]],
                                ti_v.at[jnp.int32(j)], sem)
               for j in range(NCH)]
        for cp in cps:
            cp.wait()

        for g in range(GRP):
            c = jnp.int32(g // (CHUNK // L))
            r = pl.ds(jnp.int32((g % (CHUNK // L)) * L), L)
            sidx_v[c, r] = jnp.maximum(ti_v[c, r], 0)

        cps = []
        for j in range(V):
            j32 = jnp.int32(j)
            for c in range(NCH):
                c32 = jnp.int32(c)
                cps.append(pltpu.async_copy(
                    wclt_hbm.at[j32].at[sidx_v.at[c32]],
                    cols_v.at[j32, c32], sem))
        for cp in cps:
            cp.wait()

        for g in range(GRP):
            c = jnp.int32(g // (CHUNK // L))
            r = pl.ds(jnp.int32((g % (CHUNK // L)) * L), L)
            tok = tok_v[c, r]
            idx = ti_v[c, r]
            best = tok
            for j in range(V - 1, -1, -1):
                v = cols_v[jnp.int32(j), c, r]
                best = jnp.where(v != 0, v, best)
            out_v[c, r] = jnp.where(idx >= 0, best, tok)

        pltpu.sync_copy(out_v, out_hbm.at[wid])

    return sc_kernel


def kernel(input_text, token_to_idx, word_casing_lookup):
    tok32 = input_text.astype(jnp.int32).reshape(NW, NCH, CHUNK)
    tti32 = token_to_idx.astype(jnp.int32)
    wclt32 = jnp.transpose(lax.bitcast_convert_type(word_casing_lookup, jnp.int32)[:, :, 0])
    out32 = _build()(tok32, tti32, wclt32)
    return out32.reshape(B).astype(input_text.dtype)


# speedup vs baseline: 1.6310x; 1.6310x over previous
import functools

import jax
import jax.numpy as jnp
from jax import lax
from jax.experimental import pallas as pl
from jax.experimental.pallas import tpu as pltpu
from jax.experimental.pallas import tpu_sc as plsc

B = 16384
V = 8
L = 16
NC, NS = 2, 16
NW = NC * NS
BPW = B // NW
CHUNK = 128
NCH = BPW // CHUNK
GRP = BPW // L


@functools.lru_cache(maxsize=1)
def _build():
    mesh = plsc.VectorSubcoreMesh(
        core_axis_name="c", subcore_axis_name="s", num_cores=NC, num_subcores=NS
    )

    @functools.partial(
        pl.kernel,
        out_type=jax.ShapeDtypeStruct((NW, NCH, CHUNK), jnp.int32),
        mesh=mesh,
        scratch_types=[
            pltpu.VMEM((NCH, CHUNK), jnp.int32),
            pltpu.VMEM((NCH, CHUNK), jnp.int32),
            pltpu.VMEM((NCH, CHUNK), jnp.int32),
            pltpu.VMEM((V, NCH, CHUNK), jnp.int32),
            pltpu.VMEM((NCH, CHUNK), jnp.int32),
            pltpu.SemaphoreType.DMA,
        ],
        compiler_params=pltpu.CompilerParams(
            needs_layout_passes=False, use_tc_tiling_on_sc=False,
            disable_bounds_checks=True, disable_semaphore_checks=True
        ),
    )
    def sc_kernel(tok_hbm, tti_hbm, wclt_hbm, out_hbm,
                  tok_v, ti_v, sidx_v, cols_v, out_v, sem):
        wid = lax.axis_index("s") * NC + lax.axis_index("c")
        pltpu.sync_copy(tok_hbm.at[wid], tok_v)

        cps = [pltpu.async_copy(tti_hbm.at[tok_v.at[jnp.int32(j)]],
                                ti_v.at[jnp.int32(j)], sem)
               for j in range(NCH)]
        for cp in cps:
            cp.wait()

        for g in range(GRP):
            c = jnp.int32(g // (CHUNK // L))
            r = pl.ds(jnp.int32((g % (CHUNK // L)) * L), L)
            sidx_v[c, r] = jnp.maximum(ti_v[c, r], 0)

        cps = []
        for j in range(V):
            j32 = jnp.int32(j)
            for c in range(NCH):
                c32 = jnp.int32(c)
                cps.append(pltpu.async_copy(
                    wclt_hbm.at[j32].at[sidx_v.at[c32]],
                    cols_v.at[j32, c32], sem))
        for cp in cps:
            cp.wait()

        for g in range(GRP):
            c = jnp.int32(g // (CHUNK // L))
            r = pl.ds(jnp.int32((g % (CHUNK // L)) * L), L)
            tok = tok_v[c, r]
            idx = ti_v[c, r]
            best = tok
            for j in range(V - 1, -1, -1):
                v = cols_v[jnp.int32(j), c, r]
                best = jnp.where(v != 0, v, best)
            out_v[c, r] = jnp.where(idx >= 0, best, tok)

        pltpu.sync_copy(out_v, out_hbm.at[wid])

    return sc_kernel


def kernel(input_text, token_to_idx, word_casing_lookup):
    tok32 = input_text.astype(jnp.int32).reshape(NW, NCH, CHUNK)
    tti32 = token_to_idx.astype(jnp.int32)
    wclt32 = jnp.transpose(word_casing_lookup).astype(jnp.int32)
    out32 = _build()(tok32, tti32, wclt32)
    return out32.reshape(B).astype(input_text.dtype)
